# baseline (speedup 1.0000x reference)
import functools

import jax
import jax.numpy as jnp
from jax import lax
from jax.experimental import pallas as pl
from jax.experimental.pallas import tpu as pltpu
from jax.experimental.pallas import tpu_sc as plsc

N = 10000
R = 16
NB = 8
E = 320000
D = 128

NCORES = 2
NSUB = 16
CH = 240
NCHUNK = 42
NCC = NCHUNK // NCORES
NPAD = NCHUNK * CH
CROWS = CH * R
DUMP = CROWS
AGG_ROWS = CROWS + 128
RPT = AGG_ROWS // NSUB
WPT = CROWS // NSUB
HROWS = 10240
HPT = HROWS // NSUB
EPT = E // NSUB
NV = EPT // 16
QB = 128
REGCAP = 22784
OFFW = 32

_SC_PARAMS = dict(
    mesh=plsc.VectorSubcoreMesh(core_axis_name="c", subcore_axis_name="s"),
    compiler_params=pltpu.CompilerParams(needs_layout_passes=False),
)


def _off_set(offv, lc, val):
    half, idx = divmod(lc, 16)
    lanes = lax.iota(jnp.int32, 16)
    offv[pl.ds(half * 16, 16)] = jnp.where(
        lanes == idx, val, offv[pl.ds(half * 16, 16)])


def _off_get(offv, lc):
    half, idx = divmod(lc, 16)
    return offv[pl.ds(half * 16, 16)][idx]


@functools.partial(
    pl.kernel,
    out_type=(
        jax.ShapeDtypeStruct((NCORES * NSUB * REGCAP,), jnp.int32),
        jax.ShapeDtypeStruct((NCORES * NSUB * REGCAP,), jnp.int32),
        jax.ShapeDtypeStruct((NCORES * NSUB * OFFW,), jnp.int32),
    ),
    scratch_types=[
        pltpu.VMEM((EPT,), jnp.int32),
        pltpu.VMEM((EPT,), jnp.int32),
        pltpu.VMEM((EPT,), jnp.int32),
        pltpu.VMEM((QB,), jnp.int32),
        pltpu.VMEM((QB,), jnp.int32),
        pltpu.VMEM((OFFW,), jnp.int32),
        pltpu.SMEM((2,), jnp.int32),
    ],
    **_SC_PARAMS,
)
def _sc_build_lists(src_hbm, dst_hbm, et_hbm, lsrc_hbm, lseg_hbm, off_hbm,
                    srcv, dstv, etv, qsrc, qseg, offv, cnt):
    c = lax.axis_index("c")
    s = lax.axis_index("s")
    ebase = pl.multiple_of(s * EPT, 8)
    regbase = pl.multiple_of((c * NSUB + s) * REGCAP, 128)

    pltpu.sync_copy(src_hbm.at[pl.ds(ebase, EPT)], srcv)
    pltpu.sync_copy(dst_hbm.at[pl.ds(ebase, EPT)], dstv)
    pltpu.sync_copy(et_hbm.at[pl.ds(ebase, EPT)], etv)

    def _reset_queue():
        for k in range(QB // 16):
            qsrc[pl.ds(k * 16, 16)] = jnp.zeros((16,), jnp.int32)
            qseg[pl.ds(k * 16, 16)] = jnp.full((16,), DUMP, jnp.int32)
        cnt[0] = 0

    def _flush():
        at = pl.multiple_of(regbase + cnt[1], 128)
        pltpu.sync_copy(qsrc, lsrc_hbm.at[pl.ds(at, QB)])
        pltpu.sync_copy(qseg, lseg_hbm.at[pl.ds(at, QB)])
        cnt[1] = cnt[1] + QB
        _reset_queue()

    _reset_queue()
    cnt[1] = 0
    for k in range(OFFW // 16):
        offv[pl.ds(k * 16, 16)] = jnp.zeros((16,), jnp.int32)

    for lc in range(NCC):
        base = (lc * NCORES + c) * CH
        _off_set(offv, lc, cnt[1])

        def scan_body(j, carry):
            dv = dstv[pl.ds(j * 16, 16)]
            rel = dv - base
            m = (rel >= 0) & (rel < CH)

            @pl.when(jnp.any(m))
            def _append():
                @pl.when(cnt[0] > QB - 16)
                def _maybe_flush():
                    _flush()
                ev = etv[pl.ds(j * 16, 16)]
                sv = srcv[pl.ds(j * 16, 16)]
                segv = rel * R + ev
                mi = m.astype(jnp.int32)
                pos = cnt[0] + plsc.cumsum(mi) - 1
                plsc.store_scatter(qsrc, [pos], sv, mask=m)
                plsc.store_scatter(qseg, [pos], segv, mask=m)
                cnt[0] = cnt[0] + jnp.sum(mi)
            return carry

        lax.fori_loop(0, NV, scan_body, 0)

        @pl.when(cnt[0] > 0)
        def _final_flush():
            _flush()

    _off_set(offv, NCC, cnt[1])
    pltpu.sync_copy(
        offv, off_hbm.at[pl.ds(pl.multiple_of((c * NSUB + s) * OFFW, 8), OFFW)])


@functools.partial(
    pl.kernel,
    out_type=jax.ShapeDtypeStruct((NPAD * R, D), jnp.float32),
    scratch_types=[
        pltpu.VMEM((QB,), jnp.int32),
        pltpu.VMEM((QB,), jnp.int32),
        pltpu.VMEM((QB, D), jnp.float32),
        pltpu.VMEM((OFFW,), jnp.int32),
        pltpu.VMEM_SHARED((HROWS, D), jnp.float32),
        pltpu.VMEM_SHARED((AGG_ROWS, D), jnp.float32),
        pltpu.SemaphoreType.DMA,
        pltpu.SemaphoreType.DMA,
    ],
    **_SC_PARAMS,
)
def _sc_segment_sum(lsrc_hbm, lseg_hbm, off_hbm, h_hbm, agg_hbm,
                    srcb, segb, rows, offv, h_sh, agg_sh, sem, semi):
    c = lax.axis_index("c")
    s = lax.axis_index("s")
    regbase = pl.multiple_of((c * NSUB + s) * REGCAP, 128)

    pltpu.sync_copy(
        off_hbm.at[pl.ds(pl.multiple_of((c * NSUB + s) * OFFW, 8), OFFW)], offv)
    for hh in range(5):
        hat = pl.multiple_of(s * HPT + hh * 128, 128)
        pltpu.sync_copy(h_hbm.at[pl.ds(hat, 128)], h_sh.at[pl.ds(hat, 128)])

    for lc in range(NCC):
        kc = lc * NCORES + c

        def zb(i, carry):
            rows[i, pl.ds(0, 16)] = jnp.zeros((16,), jnp.float32)
            rows[i, pl.ds(16, 16)] = jnp.zeros((16,), jnp.float32)
            rows[i, pl.ds(32, 16)] = jnp.zeros((16,), jnp.float32)
            rows[i, pl.ds(48, 16)] = jnp.zeros((16,), jnp.float32)
            rows[i, pl.ds(64, 16)] = jnp.zeros((16,), jnp.float32)
            rows[i, pl.ds(80, 16)] = jnp.zeros((16,), jnp.float32)
            rows[i, pl.ds(96, 16)] = jnp.zeros((16,), jnp.float32)
            rows[i, pl.ds(112, 16)] = jnp.zeros((16,), jnp.float32)
            return carry
        lax.fori_loop(0, QB, zb, 0)
        pltpu.sync_copy(rows,
                        agg_sh.at[pl.ds(pl.multiple_of(s * RPT, 8), 128)])
        pltpu.sync_copy(rows.at[pl.ds(0, RPT - 128)],
                        agg_sh.at[pl.ds(pl.multiple_of(s * RPT + 128, 8),
                                        RPT - 128)])
        plsc.subcore_barrier()

        start = _off_get(offv, lc)
        nblk = (_off_get(offv, lc + 1) - start) // QB

        def blk_body(b, carry):
            at = pl.multiple_of(regbase + start + b * QB, 128)
            pltpu.async_copy(lsrc_hbm.at[pl.ds(at, QB)], srcb, semi)
            pltpu.async_copy(lseg_hbm.at[pl.ds(at, QB)], segb, semi)
            pltpu.make_async_copy(lsrc_hbm.at[pl.ds(at, QB)], srcb, semi).wait()
            pltpu.make_async_copy(lseg_hbm.at[pl.ds(at, QB)], segb, semi).wait()
            pltpu.async_copy(h_sh.at[srcb], rows, sem).wait()
            pltpu.sync_copy(rows, agg_sh.at[segb], add=True)
            return carry

        lax.fori_loop(0, nblk, blk_body, 0)
        plsc.subcore_barrier()

        pltpu.sync_copy(
            agg_sh.at[pl.ds(pl.multiple_of(s * WPT, 8), WPT)],
            agg_hbm.at[pl.ds(pl.multiple_of(kc * CROWS + s * WPT, 8), WPT)])
        plsc.subcore_barrier()


def _basis_weights(coef, basesf):
    coefp = jnp.pad(coef, ((0, 0), (0, 128 - NB)))
    basesp = jnp.pad(basesf, ((0, 128 - NB), (0, 0)))

    def body(c_ref, b_ref, o_ref):
        o_ref[...] = jnp.dot(c_ref[...], b_ref[...],
                             preferred_element_type=jnp.float32)

    wt = pl.pallas_call(
        body,
        grid=(8,),
        in_specs=[
            pl.BlockSpec((R, 128), lambda i: (0, 0)),
            pl.BlockSpec((128, D * D // 8), lambda i: (0, i)),
        ],
        out_specs=pl.BlockSpec((R, D * D // 8), lambda i: (0, i)),
        out_shape=jax.ShapeDtypeStruct((R, D * D), jnp.float32),
    )(coefp, basesp)
    return wt.reshape(R * D, D)


def _dense(aggf, h, wflat, wself, bias2d, act):
    BN = 1000

    def body(a_ref, h_ref, w_ref, ws_ref, b_ref, o_ref):
        acc = jnp.dot(a_ref[...], w_ref[...], preferred_element_type=jnp.float32)
        acc = acc + jnp.dot(h_ref[...], ws_ref[...],
                            preferred_element_type=jnp.float32)
        acc = acc + b_ref[...]
        if act:
            acc = jnp.maximum(acc, 0.0)
        o_ref[...] = acc

    return pl.pallas_call(
        body,
        grid=(N // BN,),
        in_specs=[
            pl.BlockSpec((BN, R * D), lambda i: (i, 0)),
            pl.BlockSpec((BN, D), lambda i: (i, 0)),
            pl.BlockSpec((R * D, D), lambda i: (0, 0)),
            pl.BlockSpec((D, D), lambda i: (0, 0)),
            pl.BlockSpec((1, D), lambda i: (0, 0)),
        ],
        out_specs=pl.BlockSpec((BN, D), lambda i: (i, 0)),
        out_shape=jax.ShapeDtypeStruct((N, D), jnp.float32),
    )(aggf, h, wflat, wself, bias2d)


def kernel(edge_index, etypes, emb,
           bases0, coef0, wself0, bias0,
           bases1, coef1, wself1, bias1,
           bases2, coef2, wself2, bias2):
    src = edge_index[0].astype(jnp.int32)
    dst = edge_index[1].astype(jnp.int32)
    et = etypes.astype(jnp.int32)

    lsrc, lseg, off = _sc_build_lists(src, dst, et)

    h = emb
    layers = [
        (bases0, coef0, wself0, bias0, True),
        (bases1, coef1, wself1, bias1, True),
        (bases2, coef2, wself2, bias2, False),
    ]
    for bases, coef, wself, bias, act in layers:
        hp = jnp.pad(h, ((0, HROWS - N), (0, 0)))
        agg = _sc_segment_sum(lsrc, lseg, off, hp)
        wflat = _basis_weights(coef, bases.reshape(NB, D * D))
        aggf = agg[: N * R].reshape(N, R * D)
        h = _dense(aggf, h, wflat, wself, bias.reshape(1, D), act)
    return h


# speedup vs baseline: 1.2375x; 1.2375x over previous
import functools

import jax
import jax.numpy as jnp
from jax import lax
from jax.experimental import pallas as pl
from jax.experimental.pallas import tpu as pltpu
from jax.experimental.pallas import tpu_sc as plsc

N = 10000
R = 16
NB = 8
E = 320000
D = 128

NCORES = 2
NSUB = 16
CH = 240
NCHUNK = 42
NCC = NCHUNK // NCORES
NPAD = NCHUNK * CH
CROWS = CH * R
DUMP = CROWS
AGG_ROWS = CROWS + 128
RPT = AGG_ROWS // NSUB
WPT = CROWS // NSUB
HROWS = 10240
HPT = HROWS // NSUB
EPT = E // NSUB
NV = EPT // 16
QB = 128
REGCAP = 22784
OFFW = 32

_SC_PARAMS = dict(
    mesh=plsc.VectorSubcoreMesh(core_axis_name="c", subcore_axis_name="s"),
    compiler_params=pltpu.CompilerParams(needs_layout_passes=False),
)


def _off_set(offv, lc, val):
    half, idx = divmod(lc, 16)
    lanes = lax.iota(jnp.int32, 16)
    offv[pl.ds(half * 16, 16)] = jnp.where(
        lanes == idx, val, offv[pl.ds(half * 16, 16)])


def _off_get(offv, lc):
    half, idx = divmod(lc, 16)
    return offv[pl.ds(half * 16, 16)][idx]


@functools.partial(
    pl.kernel,
    out_type=(
        jax.ShapeDtypeStruct((NCORES * NSUB * REGCAP,), jnp.int32),
        jax.ShapeDtypeStruct((NCORES * NSUB * REGCAP,), jnp.int32),
        jax.ShapeDtypeStruct((NCORES * NSUB * OFFW,), jnp.int32),
    ),
    scratch_types=[
        pltpu.VMEM((EPT,), jnp.int32),
        pltpu.VMEM((EPT,), jnp.int32),
        pltpu.VMEM((EPT,), jnp.int32),
        pltpu.VMEM((EPT,), jnp.int32),
        pltpu.VMEM((EPT,), jnp.int32),
        pltpu.VMEM((QB,), jnp.int32),
        pltpu.VMEM((QB,), jnp.int32),
        pltpu.VMEM((OFFW,), jnp.int32),
        pltpu.SMEM((3,), jnp.int32),
    ],
    **_SC_PARAMS,
)
def _sc_build_lists(src_hbm, dst_hbm, et_hbm, lsrc_hbm, lseg_hbm, off_hbm,
                    srcv, dstv, etv, svc, gsegc, qsrc, qseg, offv, cnt):
    c = lax.axis_index("c")
    s = lax.axis_index("s")
    ebase = pl.multiple_of(s * EPT, 8)
    regbase = pl.multiple_of((c * NSUB + s) * REGCAP, 128)
    lanes16 = lax.iota(jnp.int32, 16)

    pltpu.sync_copy(src_hbm.at[pl.ds(ebase, EPT)], srcv)
    pltpu.sync_copy(dst_hbm.at[pl.ds(ebase, EPT)], dstv)
    pltpu.sync_copy(et_hbm.at[pl.ds(ebase, EPT)], etv)

    cnt[2] = 0

    def split_body(j, carry):
        dv = dstv[pl.ds(j * 16, 16)]
        kcv = jax.lax.shift_right_logical(dv * 17477, 22)
        m = (kcv & 1) == c
        sv = srcv[pl.ds(j * 16, 16)]
        gs = dv * R + etv[pl.ds(j * 16, 16)]
        mi = m.astype(jnp.int32)
        pos = cnt[2] + plsc.cumsum(mi) - 1
        plsc.store_scatter(svc, [pos], sv, mask=m)
        plsc.store_scatter(gsegc, [pos], gs, mask=m)
        cnt[2] = cnt[2] + jnp.sum(mi)
        return carry

    lax.fori_loop(0, NV, split_body, 0)
    nloc = cnt[2]
    nvloc = (nloc + 15) // 16

    def _reset_queue():
        for k in range(QB // 16):
            qsrc[pl.ds(k * 16, 16)] = jnp.zeros((16,), jnp.int32)
            qseg[pl.ds(k * 16, 16)] = jnp.full((16,), DUMP, jnp.int32)
        cnt[0] = 0

    def _flush():
        at = pl.multiple_of(regbase + cnt[1], 128)
        pltpu.sync_copy(qsrc, lsrc_hbm.at[pl.ds(at, QB)])
        pltpu.sync_copy(qseg, lseg_hbm.at[pl.ds(at, QB)])
        cnt[1] = cnt[1] + QB
        _reset_queue()

    _reset_queue()
    cnt[1] = 0
    for k in range(OFFW // 16):
        offv[pl.ds(k * 16, 16)] = jnp.zeros((16,), jnp.int32)

    for lc in range(NCC):
        gbase = (lc * NCORES + c) * CH * R
        _off_set(offv, lc, cnt[1])

        def scan_body(j, carry):
            gs = gsegc[pl.ds(j * 16, 16)]
            segv = gs - gbase
            m = (segv >= 0) & (segv < CH * R) & (j * 16 + lanes16 < nloc)

            @pl.when(jnp.any(m))
            def _append():
                @pl.when(cnt[0] > QB - 16)
                def _maybe_flush():
                    _flush()
                sv = svc[pl.ds(j * 16, 16)]
                mi = m.astype(jnp.int32)
                pos = cnt[0] + plsc.cumsum(mi) - 1
                plsc.store_scatter(qsrc, [pos], sv, mask=m)
                plsc.store_scatter(qseg, [pos], segv, mask=m)
                cnt[0] = cnt[0] + jnp.sum(mi)
            return carry

        lax.fori_loop(0, nvloc, scan_body, 0)

        @pl.when(cnt[0] > 0)
        def _final_flush():
            _flush()

    _off_set(offv, NCC, cnt[1])
    pltpu.sync_copy(
        offv, off_hbm.at[pl.ds(pl.multiple_of((c * NSUB + s) * OFFW, 8), OFFW)])


@functools.partial(
    pl.kernel,
    out_type=jax.ShapeDtypeStruct((NPAD * R, D), jnp.float32),
    scratch_types=[
        pltpu.VMEM((QB,), jnp.int32),
        pltpu.VMEM((QB,), jnp.int32),
        pltpu.VMEM((QB, D), jnp.float32),
        pltpu.VMEM((OFFW,), jnp.int32),
        pltpu.VMEM_SHARED((HROWS, D), jnp.float32),
        pltpu.VMEM_SHARED((AGG_ROWS, D), jnp.float32),
        pltpu.SemaphoreType.DMA,
        pltpu.SemaphoreType.DMA,
    ],
    **_SC_PARAMS,
)
def _sc_segment_sum(lsrc_hbm, lseg_hbm, off_hbm, h_hbm, agg_hbm,
                    srcb, segb, rows, offv, h_sh, agg_sh, sem, semi):
    c = lax.axis_index("c")
    s = lax.axis_index("s")
    regbase = pl.multiple_of((c * NSUB + s) * REGCAP, 128)

    pltpu.sync_copy(
        off_hbm.at[pl.ds(pl.multiple_of((c * NSUB + s) * OFFW, 8), OFFW)], offv)
    for hh in range(5):
        hat = pl.multiple_of(s * HPT + hh * 128, 128)
        pltpu.sync_copy(h_hbm.at[pl.ds(hat, 128)], h_sh.at[pl.ds(hat, 128)])

    for lc in range(NCC):
        kc = lc * NCORES + c

        def zb(i, carry):
            rows[i, pl.ds(0, 16)] = jnp.zeros((16,), jnp.float32)
            rows[i, pl.ds(16, 16)] = jnp.zeros((16,), jnp.float32)
            rows[i, pl.ds(32, 16)] = jnp.zeros((16,), jnp.float32)
            rows[i, pl.ds(48, 16)] = jnp.zeros((16,), jnp.float32)
            rows[i, pl.ds(64, 16)] = jnp.zeros((16,), jnp.float32)
            rows[i, pl.ds(80, 16)] = jnp.zeros((16,), jnp.float32)
            rows[i, pl.ds(96, 16)] = jnp.zeros((16,), jnp.float32)
            rows[i, pl.ds(112, 16)] = jnp.zeros((16,), jnp.float32)
            return carry
        lax.fori_loop(0, QB, zb, 0)
        pltpu.sync_copy(rows,
                        agg_sh.at[pl.ds(pl.multiple_of(s * RPT, 8), 128)])
        pltpu.sync_copy(rows.at[pl.ds(0, RPT - 128)],
                        agg_sh.at[pl.ds(pl.multiple_of(s * RPT + 128, 8),
                                        RPT - 128)])
        plsc.subcore_barrier()

        start = _off_get(offv, lc)
        nblk = (_off_get(offv, lc + 1) - start) // QB

        def blk_body(b, carry):
            at = pl.multiple_of(regbase + start + b * QB, 128)
            pltpu.async_copy(lsrc_hbm.at[pl.ds(at, QB)], srcb, semi)
            pltpu.async_copy(lseg_hbm.at[pl.ds(at, QB)], segb, semi)
            pltpu.make_async_copy(lsrc_hbm.at[pl.ds(at, QB)], srcb, semi).wait()
            pltpu.make_async_copy(lseg_hbm.at[pl.ds(at, QB)], segb, semi).wait()
            pltpu.async_copy(h_sh.at[srcb], rows, sem).wait()
            pltpu.sync_copy(rows, agg_sh.at[segb], add=True)
            return carry

        lax.fori_loop(0, nblk, blk_body, 0)
        plsc.subcore_barrier()

        pltpu.sync_copy(
            agg_sh.at[pl.ds(pl.multiple_of(s * WPT, 8), WPT)],
            agg_hbm.at[pl.ds(pl.multiple_of(kc * CROWS + s * WPT, 8), WPT)])
        plsc.subcore_barrier()


def _basis_weights(coef, basesf):
    coefp = jnp.pad(coef, ((0, 0), (0, 128 - NB)))
    basesp = jnp.pad(basesf, ((0, 128 - NB), (0, 0)))

    def body(c_ref, b_ref, o_ref):
        o_ref[...] = jnp.dot(c_ref[...], b_ref[...],
                             preferred_element_type=jnp.float32)

    wt = pl.pallas_call(
        body,
        grid=(8,),
        in_specs=[
            pl.BlockSpec((R, 128), lambda i: (0, 0)),
            pl.BlockSpec((128, D * D // 8), lambda i: (0, i)),
        ],
        out_specs=pl.BlockSpec((R, D * D // 8), lambda i: (0, i)),
        out_shape=jax.ShapeDtypeStruct((R, D * D), jnp.float32),
    )(coefp, basesp)
    return wt.reshape(R * D, D)


def _dense(aggf, h, wflat, wself, bias2d, act):
    BN = 1000

    def body(a_ref, h_ref, w_ref, ws_ref, b_ref, o_ref):
        acc = jnp.dot(a_ref[...], w_ref[...], preferred_element_type=jnp.float32)
        acc = acc + jnp.dot(h_ref[...], ws_ref[...],
                            preferred_element_type=jnp.float32)
        acc = acc + b_ref[...]
        if act:
            acc = jnp.maximum(acc, 0.0)
        o_ref[...] = acc

    return pl.pallas_call(
        body,
        grid=(N // BN,),
        in_specs=[
            pl.BlockSpec((BN, R * D), lambda i: (i, 0)),
            pl.BlockSpec((BN, D), lambda i: (i, 0)),
            pl.BlockSpec((R * D, D), lambda i: (0, 0)),
            pl.BlockSpec((D, D), lambda i: (0, 0)),
            pl.BlockSpec((1, D), lambda i: (0, 0)),
        ],
        out_specs=pl.BlockSpec((BN, D), lambda i: (i, 0)),
        out_shape=jax.ShapeDtypeStruct((N, D), jnp.float32),
    )(aggf, h, wflat, wself, bias2d)


def kernel(edge_index, etypes, emb,
           bases0, coef0, wself0, bias0,
           bases1, coef1, wself1, bias1,
           bases2, coef2, wself2, bias2):
    src = edge_index[0].astype(jnp.int32)
    dst = edge_index[1].astype(jnp.int32)
    et = etypes.astype(jnp.int32)

    lsrc, lseg, off = _sc_build_lists(src, dst, et)

    h = emb
    layers = [
        (bases0, coef0, wself0, bias0, True),
        (bases1, coef1, wself1, bias1, True),
        (bases2, coef2, wself2, bias2, False),
    ]
    for bases, coef, wself, bias, act in layers:
        hp = jnp.pad(h, ((0, HROWS - N), (0, 0)))
        agg = _sc_segment_sum(lsrc, lseg, off, hp)
        wflat = _basis_weights(coef, bases.reshape(NB, D * D))
        aggf = agg[: N * R].reshape(N, R * D)
        h = _dense(aggf, h, wflat, wself, bias.reshape(1, D), act)
    return h
